# baseline (speedup 1.0000x reference)
import functools

import jax
import jax.numpy as jnp
from jax import lax
from jax.experimental import pallas as pl
from jax.experimental.pallas import tpu as pltpu
from jax.experimental.pallas import tpu_sc as plsc


NC = 2
NS = 16
NW = NC * NS

CHUNK = 32


def _emb_body(nchunk, rows_per_tile, seq, d, x_hbm, tok_hbm, pos_hbm,
              out_hbm, idx_v, tok_v, pos_v, sem):
    wid = lax.axis_index("s") * NC + lax.axis_index("c")
    base = wid * rows_per_tile
    s0 = lax.rem(base, seq)
    pltpu.sync_copy(x_hbm.at[pl.ds(base, rows_per_tile)], idx_v)

    nvec = d // 16

    for j in range(nchunk):
        cp = pltpu.async_copy(tok_hbm.at[idx_v.at[pl.ds(j * CHUNK, CHUNK)]],
                              tok_v, sem)
        pltpu.sync_copy(pos_hbm.at[pl.ds(s0 + j * CHUNK, CHUNK)], pos_v)
        cp.wait()

        def add_row(r, carry):
            def add_vec(c, carry2):
                off = c * 16
                tok_v[r, pl.ds(off, 16)] = (tok_v[r, pl.ds(off, 16)]
                                            + pos_v[r, pl.ds(off, 16)])
                return carry2
            return lax.fori_loop(0, nvec, add_vec, carry)

        lax.fori_loop(0, CHUNK, add_row, 0)

        pltpu.sync_copy(tok_v, out_hbm.at[pl.ds(base + j * CHUNK, CHUNK)])


def kernel(x, token_table, position_table):
    b, seq = x.shape
    d = token_table.shape[1]
    tot = b * seq
    rows_per_tile = tot // NW
    nchunk = rows_per_tile // CHUNK

    x_flat = x.reshape(tot).astype(jnp.int32)

    mesh = plsc.VectorSubcoreMesh(core_axis_name="c", subcore_axis_name="s")
    emb = pl.kernel(
        functools.partial(_emb_body, nchunk, rows_per_tile, seq, d),
        out_type=jax.ShapeDtypeStruct((tot, d), jnp.float32),
        mesh=mesh,
        scratch_types=[
            pltpu.VMEM((rows_per_tile,), jnp.int32),
            pltpu.VMEM((CHUNK, d), jnp.float32),
            pltpu.VMEM((CHUNK, d), jnp.float32),
            pltpu.SemaphoreType.DMA,
        ],
    )
    out = emb(x_flat, token_table, position_table)
    return out.reshape(b, seq, d)


# speedup vs baseline: 2.7765x; 2.7765x over previous
import functools

import jax
import jax.numpy as jnp
from jax import lax
from jax.experimental import pallas as pl
from jax.experimental.pallas import tpu as pltpu
from jax.experimental.pallas import tpu_sc as plsc


NC = 2
NS = 16
NW = NC * NS

CHUNK = 32
UNROLL = 4


def _emb_body(nb, seq, s_per_tile, d, x_hbm, tok_hbm, pos_hbm, out_hbm,
              idx_v, tok0, tok1, pos_v, gsem, wsem):
    wid = lax.axis_index("s") * NC + lax.axis_index("c")
    s_base = wid * s_per_tile
    nq = s_per_tile // CHUNK
    nvec = d // 16

    for b in range(nb):
        pltpu.sync_copy(x_hbm.at[pl.ds(b * seq + s_base, s_per_tile)],
                        idx_v.at[pl.ds(b * s_per_tile, s_per_tile)])

    toks = (tok0, tok1)
    nstep = nq * nb

    def gather(t):
        q, b = divmod(t, nb)
        src = tok_hbm.at[idx_v.at[pl.ds(b * s_per_tile + q * CHUNK, CHUNK)]]
        return pltpu.async_copy(src, toks[t % 2], gsem)

    writes = [None] * nstep
    g_next = gather(0)
    pltpu.sync_copy(pos_hbm.at[pl.ds(s_base, CHUNK)], pos_v)

    for t in range(nstep):
        q, b = divmod(t, nb)
        g_cur = g_next
        if t + 1 < nstep:
            if t >= 1:
                writes[t - 1].wait()
            g_next = gather(t + 1)
        g_cur.wait()

        tok = toks[t % 2]

        def add_row(r, carry):
            def add_vec(c, carry2):
                for u in range(UNROLL):
                    o = c * (16 * UNROLL) + u * 16
                    plsc.addupdate(tok.at[r, pl.ds(o, 16)],
                                   pos_v[r, pl.ds(o, 16)])
                return carry2
            return lax.fori_loop(0, nvec // UNROLL, add_vec, carry)

        lax.fori_loop(0, CHUNK, add_row, 0)

        row0 = b * seq + s_base + q * CHUNK
        writes[t] = pltpu.async_copy(tok, out_hbm.at[pl.ds(row0, CHUNK)],
                                     wsem)
        if b == nb - 1 and q + 1 < nq:
            pltpu.sync_copy(pos_hbm.at[pl.ds(s_base + (q + 1) * CHUNK, CHUNK)],
                            pos_v)

    writes[nstep - 2].wait()
    writes[nstep - 1].wait()


def kernel(x, token_table, position_table):
    nb, seq = x.shape
    d = token_table.shape[1]
    tot = nb * seq
    s_per_tile = seq // NW

    x_flat = x.reshape(tot).astype(jnp.int32)

    mesh = plsc.VectorSubcoreMesh(core_axis_name="c", subcore_axis_name="s")
    emb = pl.kernel(
        functools.partial(_emb_body, nb, seq, s_per_tile, d),
        out_type=jax.ShapeDtypeStruct((tot, d), jnp.float32),
        mesh=mesh,
        scratch_types=[
            pltpu.VMEM((nb * s_per_tile,), jnp.int32),
            pltpu.VMEM((CHUNK, d), jnp.float32),
            pltpu.VMEM((CHUNK, d), jnp.float32),
            pltpu.VMEM((CHUNK, d), jnp.float32),
            pltpu.SemaphoreType.DMA,
            pltpu.SemaphoreType.DMA,
        ],
    )
    out = emb(x_flat, token_table, position_table)
    return out.reshape(nb, seq, d)
